# baseline (speedup 1.0000x reference)
import jax
import jax.numpy as jnp
from jax.experimental import pallas as pl
from jax.experimental.pallas import tpu as pltpu

_BT = 1024
_BF = 1024


def _moe_kernel(xb_ref, gw_ref, wg_ref, wu_ref, wd_ref, o_ref, probs_ref):
    e = pl.program_id(1)
    f = pl.program_id(2)
    num_e = pl.num_programs(1)

    first = jnp.logical_and(e == 0, f == 0)
    xb = xb_ref[...]

    @pl.when(first)
    def _():
        logits = jax.lax.dot_general(
            xb, gw_ref[...].astype(jnp.bfloat16), (((1,), (1,)), ((), ())),
            preferred_element_type=jnp.float32)
        m = jnp.max(logits, axis=1, keepdims=True)
        ex = jnp.exp(logits - m)
        probs_ref[...] = ex / jnp.sum(ex, axis=1, keepdims=True)

    sel = jax.lax.broadcasted_iota(jnp.int32, (1, num_e), 1) == e
    p = jnp.sum(jnp.where(sel, probs_ref[...], 0.0), axis=1,
                keepdims=True)

    wg = wg_ref[0].astype(jnp.bfloat16)
    wu = wu_ref[0].astype(jnp.bfloat16)
    wd = wd_ref[0].astype(jnp.bfloat16)

    g = jax.lax.dot_general(xb, wg, (((1,), (1,)), ((), ())),
                            preferred_element_type=jnp.float32)
    u = jax.lax.dot_general(xb, wu, (((1,), (1,)), ((), ())),
                            preferred_element_type=jnp.float32)
    h = (p * jax.nn.sigmoid(g) * u).astype(jnp.bfloat16)

    d_half = wd.shape[0] // 2
    for k in range(2):
        cols = slice(k * d_half, (k + 1) * d_half)
        part = jax.lax.dot_general(h, wd[cols, :], (((1,), (1,)), ((), ())),
                                   preferred_element_type=jnp.float32)

        @pl.when(first)
        def _():
            o_ref[:, cols] = part

        @pl.when(jnp.logical_not(first))
        def _():
            o_ref[:, cols] += part


def kernel(hidden_states, gate_w, w_gate, w_up, w_down):
    orig_shape = hidden_states.shape
    d = orig_shape[-1]
    x = hidden_states.reshape(-1, d)
    t = x.shape[0]
    num_e, dff, _ = w_gate.shape
    xb = x.astype(jnp.bfloat16)

    out = pl.pallas_call(
        _moe_kernel,
        grid=(t // _BT, num_e, dff // _BF),
        in_specs=[
            pl.BlockSpec((_BT, d), lambda i, e, f: (i, 0)),
            pl.BlockSpec((num_e, d), lambda i, e, f: (0, 0)),
            pl.BlockSpec((1, _BF, d), lambda i, e, f: (e, f, 0)),
            pl.BlockSpec((1, _BF, d), lambda i, e, f: (e, f, 0)),
            pl.BlockSpec((1, d, _BF), lambda i, e, f: (e, 0, f)),
        ],
        out_specs=pl.BlockSpec((_BT, d), lambda i, e, f: (i, 0)),
        out_shape=jax.ShapeDtypeStruct((t, d), x.dtype),
        scratch_shapes=[pltpu.VMEM((_BT, num_e), jnp.float32)],
        compiler_params=pltpu.CompilerParams(
            dimension_semantics=("parallel", "arbitrary", "arbitrary"),
        ),
    )(xb, gate_w, w_gate, w_up, w_down)
    return out.reshape(orig_shape)


# speedup vs baseline: 1.0500x; 1.0500x over previous
import jax
import jax.numpy as jnp
from jax.experimental import pallas as pl
from jax.experimental.pallas import tpu as pltpu

_BT = 1024
_BF = 1024


def _moe_kernel(xb_ref, gw_ref, wg_ref, wu_ref, wd_ref, o_ref, probs_ref):
    e = pl.program_id(1)
    f = pl.program_id(2)
    num_e = pl.num_programs(1)

    first = jnp.logical_and(e == 0, f == 0)
    xb = xb_ref[...]

    @pl.when(first)
    def _():
        logits = jax.lax.dot_general(
            xb, gw_ref[...].astype(jnp.bfloat16), (((1,), (1,)), ((), ())),
            preferred_element_type=jnp.float32)
        m = jnp.max(logits, axis=1, keepdims=True)
        ex = jnp.exp(logits - m)
        probs_ref[...] = ex / jnp.sum(ex, axis=1, keepdims=True)

    sel = jax.lax.broadcasted_iota(jnp.int32, (1, num_e), 1) == e
    p = jnp.sum(jnp.where(sel, probs_ref[...], 0.0), axis=1,
                keepdims=True)

    wg = wg_ref[0].astype(jnp.bfloat16)
    wu = wu_ref[0].astype(jnp.bfloat16)
    wd = wd_ref[0].astype(jnp.bfloat16)

    g = jax.lax.dot_general(xb, wg, (((1,), (1,)), ((), ())),
                            preferred_element_type=jnp.float32)
    u = jax.lax.dot_general(xb, wu, (((1,), (1,)), ((), ())),
                            preferred_element_type=jnp.float32)
    h = (p * jax.nn.sigmoid(g) * u).astype(jnp.bfloat16)

    part = jax.lax.dot_general(h, wd, (((1,), (1,)), ((), ())),
                               preferred_element_type=jnp.float32)

    @pl.when(first)
    def _():
        o_ref[...] = part

    @pl.when(jnp.logical_not(first))
    def _():
        o_ref[...] += part


def kernel(hidden_states, gate_w, w_gate, w_up, w_down):
    orig_shape = hidden_states.shape
    d = orig_shape[-1]
    x = hidden_states.reshape(-1, d)
    t = x.shape[0]
    num_e, dff, _ = w_gate.shape
    xb = x.astype(jnp.bfloat16)

    out = pl.pallas_call(
        _moe_kernel,
        grid=(t // _BT, num_e, dff // _BF),
        in_specs=[
            pl.BlockSpec((_BT, d), lambda i, e, f: (i, 0)),
            pl.BlockSpec((num_e, d), lambda i, e, f: (0, 0)),
            pl.BlockSpec((1, _BF, d), lambda i, e, f: (e, f, 0)),
            pl.BlockSpec((1, _BF, d), lambda i, e, f: (e, f, 0)),
            pl.BlockSpec((1, d, _BF), lambda i, e, f: (e, 0, f)),
        ],
        out_specs=pl.BlockSpec((_BT, d), lambda i, e, f: (i, 0)),
        out_shape=jax.ShapeDtypeStruct((t, d), x.dtype),
        scratch_shapes=[pltpu.VMEM((_BT, num_e), jnp.float32)],
        compiler_params=pltpu.CompilerParams(
            dimension_semantics=("parallel", "arbitrary", "arbitrary"),
        ),
    )(xb, gate_w, w_gate, w_up, w_down)
    return out.reshape(orig_shape)
